# baseline (speedup 1.0000x reference)
import functools

import jax
import jax.numpy as jnp
from jax.experimental import pallas as pl

_S = 512
_H = 32
_D = 128
_T = 8192

_BS = 64
_BT = 512


def _body(q_ref, k_ref, w_ref, ks_ref, ke_ref, o_ref):
    j = pl.program_id(1)
    scores = jax.lax.dot_general(
        q_ref[...], k_ref[...],
        dimension_numbers=(((1,), (1,)), ((), ())),
        preferred_element_type=jnp.float32,
    )
    scores = jnp.maximum(scores, 0.0).reshape(_BS, _H, _BT)
    logits = jnp.sum(scores * w_ref[...][:, :, None], axis=1)
    t_ids = j * _BT + jax.lax.broadcasted_iota(jnp.int32, (_BS, _BT), 1)
    mask = (t_ids >= ks_ref[...]) & (t_ids < ke_ref[...])
    o_ref[...] = jnp.where(mask, logits, -jnp.inf)


@jax.jit
def kernel(index_q, index_k, weights, cu_seqlen_ks, cu_seqlen_ke):
    q2 = index_q.reshape(_S * _H, _D).astype(jnp.bfloat16)
    k2 = index_k.astype(jnp.bfloat16)
    ks2 = cu_seqlen_ks.reshape(_S, 1)
    ke2 = cu_seqlen_ke.reshape(_S, 1)
    grid = (_S // _BS, _T // _BT)
    return pl.pallas_call(
        _body,
        grid=grid,
        in_specs=[
            pl.BlockSpec((_BS * _H, _D), lambda i, j: (i, 0)),
            pl.BlockSpec((_BT, _D), lambda i, j: (j, 0)),
            pl.BlockSpec((_BS, _H), lambda i, j: (i, 0)),
            pl.BlockSpec((_BS, 1), lambda i, j: (i, 0)),
            pl.BlockSpec((_BS, 1), lambda i, j: (i, 0)),
        ],
        out_specs=pl.BlockSpec((_BS, _BT), lambda i, j: (i, j)),
        out_shape=jax.ShapeDtypeStruct((_S, _T), jnp.float32),
    )(q2, k2, weights, ks2, ke2)
